# baseline (speedup 1.0000x reference)
import functools

import jax
import jax.numpy as jnp
from jax import lax
from jax.experimental import pallas as pl
from jax.experimental.pallas import tpu as pltpu
from jax.experimental.pallas import tpu_sc as plsc

_MEMORY_COEF = 0.25
_DHAT_COEF = 0.1

_COLS = 512
_ROWS_SC = 0
_BATCHES_PER_STEP = 2
_NW = 32
_CR = 32


def _make_tc_body(rec_scale, mem_scale, batches_per_step):
    def body(xhat_ref, x_ref, h_ref, m_ref, dhat_ref, out_ref):
        i = pl.program_id(0)
        s = pl.program_id(1)

        diff = xhat_ref[...] - x_ref[...]
        rec_part = jnp.sum(diff * diff)

        @pl.when((i == 0) & (s == 0))
        def _init():
            out_ref[...] = jnp.full((1, 1), -_DHAT_COEF, jnp.float32) * jnp.mean(
                dhat_ref[...], keepdims=True
            )

        out_ref[...] += jnp.reshape(rec_part * rec_scale, (1, 1))

        @pl.when(s == 1)
        def _codebook():
            _codebook_part(h_ref, m_ref, out_ref)

    def _codebook_part(h_ref, m_ref, out_ref):
        m = m_ref[...]
        m_sq = jnp.sum(m * m, axis=0)
        pad = 8
        t_dim = h_ref.shape[2]
        m_ext = jnp.concatenate(
            [m, -0.5 * m_sq[None, :], jnp.zeros((pad - 1, m.shape[1]), m.dtype)],
            axis=0,
        ).astype(jnp.bfloat16)
        ones_row = jnp.concatenate(
            [
                jnp.ones((1, t_dim), jnp.float32),
                jnp.zeros((pad - 1, t_dim), jnp.float32),
            ],
            axis=0,
        )
        mem_part = jnp.zeros((), jnp.float32)
        for b in range(batches_per_step):
            h = h_ref[b]
            h_ext = jnp.concatenate([h, ones_row], axis=0).astype(jnp.bfloat16)
            cross = jax.lax.dot_general(
                h_ext,
                m_ext,
                (((0,), (0,)), ((), ())),
                preferred_element_type=jnp.float32,
            )
            h_sq = jnp.sum(h * h, axis=0)
            dmin = -2.0 * jnp.max(cross, axis=1)
            mem_part += jnp.sum(jnp.maximum(h_sq + dmin, 0.0))

        out_ref[...] += jnp.reshape(mem_part * mem_scale, (1, 1))

    return body


def _make_sc_kernel(row_off, rows_per_w, nchunks):
    mesh = plsc.VectorSubcoreMesh(core_axis_name="c", subcore_axis_name="s")

    @functools.partial(
        pl.kernel,
        out_type=jax.ShapeDtypeStruct((_NW * 16,), jnp.float32),
        mesh=mesh,
        scratch_types=[
            pltpu.VMEM((2, _CR, _COLS), jnp.float32),
            pltpu.VMEM((2, _CR, _COLS), jnp.float32),
            pltpu.VMEM((16,), jnp.float32),
            pltpu.SemaphoreType.DMA,
            pltpu.SemaphoreType.DMA,
            pltpu.SemaphoreType.DMA,
            pltpu.SemaphoreType.DMA,
        ],
    )
    def sc_kernel(a_hbm, b_hbm, out_hbm, abuf, bbuf, accv, sa0, sa1, sb0, sb1):
        c = lax.axis_index("c")
        s = lax.axis_index("s")
        wid = s * 2 + c
        base = row_off + wid * rows_per_w
        sems_a = (sa0, sa1)
        sems_b = (sb0, sb1)

        copies = {}

        def start(g):
            slot = g % 2
            rbase = base + g * _CR
            copies[(g, 0)] = pltpu.async_copy(
                a_hbm.at[pl.ds(rbase, _CR)], abuf.at[slot], sems_a[slot]
            )
            copies[(g, 1)] = pltpu.async_copy(
                b_hbm.at[pl.ds(rbase, _CR)], bbuf.at[slot], sems_b[slot]
            )

        start(0)
        acc = jnp.zeros((16,), jnp.float32)
        for g in range(nchunks):
            if g + 1 < nchunks:
                start(g + 1)
            copies[(g, 0)].wait()
            copies[(g, 1)].wait()
            slot = g % 2
            ar = abuf.at[slot]
            br = bbuf.at[slot]

            def row_body(r, a_carry, ar=ar, br=br):
                arr = ar.at[r]
                brr = br.at[r]

                def lane_body(j, a2):
                    av = arr[pl.ds(j * 16, 16)]
                    bv = brr[pl.ds(j * 16, 16)]
                    dd = av - bv
                    return a2 + dd * dd

                return lax.fori_loop(0, _COLS // 16, lane_body, a_carry, unroll=8)

            acc = lax.fori_loop(0, _CR, row_body, acc)

        accv[...] = acc
        pltpu.sync_copy(accv, out_hbm.at[pl.ds(wid * 16, 16)])

    return sc_kernel


def kernel(Xhat, X, H, M, Dhat):
    B, d, T = H.shape
    K = M.shape[1]
    n_rec = Xhat.size
    rows = n_rec // _COLS
    rows_tc = rows - _ROWS_SC

    Xhat2 = Xhat.reshape(rows, _COLS)
    X2 = X.reshape(rows, _COLS)
    Dhat2 = Dhat.reshape(1, B)

    rec_scale = 1.0 / float(n_rec)
    mem_scale = _MEMORY_COEF * 2.0 / float(B * d * T)
    bps = _BATCHES_PER_STEP
    nsteps = B // bps
    br = rows_tc // (nsteps * 2)

    tc_out = pl.pallas_call(
        _make_tc_body(rec_scale, mem_scale, bps),
        grid=(nsteps, 2),
        in_specs=[
            pl.BlockSpec((br, _COLS), lambda i, s: (i * 2 + s, 0)),
            pl.BlockSpec((br, _COLS), lambda i, s: (i * 2 + s, 0)),
            pl.BlockSpec((bps, d, T), lambda i, s: (i, 0, 0)),
            pl.BlockSpec((d, K), lambda i, s: (0, 0)),
            pl.BlockSpec((1, B), lambda i, s: (0, 0)),
        ],
        out_specs=pl.BlockSpec((1, 1), lambda i, s: (0, 0)),
        out_shape=jax.ShapeDtypeStruct((1, 1), jnp.float32),
    )(Xhat2, X2, H, M, Dhat2)

    if _ROWS_SC == 0:
        return tc_out[0, 0]

    rows_per_w = _ROWS_SC // _NW
    nchunks = rows_per_w // _CR
    sc_kernel = _make_sc_kernel(rows_tc, rows_per_w, nchunks)
    sc_out = sc_kernel(Xhat2, X2)

    return tc_out[0, 0] + jnp.sum(sc_out) * rec_scale


# speedup vs baseline: 1.3065x; 1.3065x over previous
import functools

import jax
import jax.numpy as jnp
from jax import lax
from jax.experimental import pallas as pl
from jax.experimental.pallas import tpu as pltpu
from jax.experimental.pallas import tpu_sc as plsc

_MEMORY_COEF = 0.25
_DHAT_COEF = 0.1

_COLS = 512
_ROWS_SC = 0
_BATCHES_PER_STEP = 2
_NW = 32
_CR = 32


def _make_tc_body(rec_scale, mem_scale, batches_per_step):
    def body(xhat_ref, x_ref, h_ref, m_ref, dhat_ref, out_ref):
        i = pl.program_id(0)

        diff = xhat_ref[...] - x_ref[...]
        rec_part = jnp.sum(diff * diff)

        m = m_ref[...]
        m_sq = jnp.sum(m * m, axis=0)
        pad = 8
        t_dim = h_ref.shape[2]
        m_ext = jnp.concatenate(
            [m, -0.5 * m_sq[None, :], jnp.zeros((pad - 1, m.shape[1]), m.dtype)],
            axis=0,
        ).astype(jnp.bfloat16)
        ones_row = jnp.concatenate(
            [
                jnp.ones((1, t_dim), jnp.float32),
                jnp.zeros((pad - 1, t_dim), jnp.float32),
            ],
            axis=0,
        )
        mem_part = jnp.zeros((), jnp.float32)
        for b in range(batches_per_step):
            h = h_ref[b]
            h_ext = jnp.concatenate([h, ones_row], axis=0).astype(jnp.bfloat16)
            cross = jax.lax.dot_general(
                h_ext,
                m_ext,
                (((0,), (0,)), ((), ())),
                preferred_element_type=jnp.float32,
            )
            h_sq = jnp.sum(h * h, axis=0)
            dmin = -2.0 * jnp.max(cross, axis=1)
            mem_part += jnp.sum(jnp.maximum(h_sq + dmin, 0.0))

        part = rec_part * rec_scale + mem_part * mem_scale

        @pl.when(i == 0)
        def _init():
            out_ref[...] = jnp.full((1, 1), -_DHAT_COEF, jnp.float32) * jnp.mean(
                dhat_ref[...], keepdims=True
            )

        out_ref[...] += jnp.reshape(part, (1, 1))

    return body


def _make_sc_kernel(row_off, rows_per_w, nchunks):
    mesh = plsc.VectorSubcoreMesh(core_axis_name="c", subcore_axis_name="s")

    @functools.partial(
        pl.kernel,
        out_type=jax.ShapeDtypeStruct((_NW * 16,), jnp.float32),
        mesh=mesh,
        scratch_types=[
            pltpu.VMEM((2, _CR, _COLS), jnp.float32),
            pltpu.VMEM((2, _CR, _COLS), jnp.float32),
            pltpu.VMEM((16,), jnp.float32),
            pltpu.SemaphoreType.DMA,
            pltpu.SemaphoreType.DMA,
            pltpu.SemaphoreType.DMA,
            pltpu.SemaphoreType.DMA,
        ],
    )
    def sc_kernel(a_hbm, b_hbm, out_hbm, abuf, bbuf, accv, sa0, sa1, sb0, sb1):
        c = lax.axis_index("c")
        s = lax.axis_index("s")
        wid = s * 2 + c
        base = row_off + wid * rows_per_w
        sems_a = (sa0, sa1)
        sems_b = (sb0, sb1)

        copies = {}

        def start(g):
            slot = g % 2
            rbase = base + g * _CR
            copies[(g, 0)] = pltpu.async_copy(
                a_hbm.at[pl.ds(rbase, _CR)], abuf.at[slot], sems_a[slot]
            )
            copies[(g, 1)] = pltpu.async_copy(
                b_hbm.at[pl.ds(rbase, _CR)], bbuf.at[slot], sems_b[slot]
            )

        start(0)
        acc = jnp.zeros((16,), jnp.float32)
        for g in range(nchunks):
            if g + 1 < nchunks:
                start(g + 1)
            copies[(g, 0)].wait()
            copies[(g, 1)].wait()
            slot = g % 2
            ar = abuf.at[slot]
            br = bbuf.at[slot]

            def row_body(r, a_carry, ar=ar, br=br):
                arr = ar.at[r]
                brr = br.at[r]

                def lane_body(j, a2):
                    av = arr[pl.ds(j * 16, 16)]
                    bv = brr[pl.ds(j * 16, 16)]
                    dd = av - bv
                    return a2 + dd * dd

                return lax.fori_loop(0, _COLS // 16, lane_body, a_carry, unroll=8)

            acc = lax.fori_loop(0, _CR, row_body, acc)

        accv[...] = acc
        pltpu.sync_copy(accv, out_hbm.at[pl.ds(wid * 16, 16)])

    return sc_kernel


def kernel(Xhat, X, H, M, Dhat):
    B, d, T = H.shape
    K = M.shape[1]
    n_rec = Xhat.size
    rows = n_rec // _COLS
    rows_tc = rows - _ROWS_SC

    Xhat2 = Xhat.reshape(rows, _COLS)
    X2 = X.reshape(rows, _COLS)
    Dhat2 = Dhat.reshape(1, B)

    rec_scale = 1.0 / float(n_rec)
    mem_scale = _MEMORY_COEF * 2.0 / float(B * d * T)
    bps = _BATCHES_PER_STEP
    nsteps = B // bps
    br = rows_tc // nsteps

    tc_out = pl.pallas_call(
        _make_tc_body(rec_scale, mem_scale, bps),
        grid=(nsteps,),
        in_specs=[
            pl.BlockSpec((br, _COLS), lambda i: (i, 0)),
            pl.BlockSpec((br, _COLS), lambda i: (i, 0)),
            pl.BlockSpec((bps, d, T), lambda i: (i, 0, 0)),
            pl.BlockSpec((d, K), lambda i: (0, 0)),
            pl.BlockSpec((1, B), lambda i: (0, 0)),
        ],
        out_specs=pl.BlockSpec((1, 1), lambda i: (0, 0)),
        out_shape=jax.ShapeDtypeStruct((1, 1), jnp.float32),
    )(Xhat2, X2, H, M, Dhat2)

    if _ROWS_SC == 0:
        return tc_out[0, 0]

    rows_per_w = _ROWS_SC // _NW
    nchunks = rows_per_w // _CR
    sc_kernel = _make_sc_kernel(rows_tc, rows_per_w, nchunks)
    sc_out = sc_kernel(Xhat2, X2)

    return tc_out[0, 0] + jnp.sum(sc_out) * rec_scale
